# Initial kernel scaffold; baseline (speedup 1.0000x reference)
import jax
import jax.numpy as jnp
from jax.experimental import pallas as pl


def kernel(x, edge_index, W1_l, b1, W1_r, W2_l, b2, W2_r):
    raise NotImplementedError("write your pallas kernel here")


# speedup vs baseline: 7.8123x; 7.8123x over previous
import functools

import jax
import jax.numpy as jnp
from jax import lax
from jax.experimental import pallas as pl
from jax.experimental.pallas import tpu as pltpu
from jax.experimental.pallas import tpu_sc as plsc

N = 10000
E = 320000
D = 128
NP = 10240

NC = 2
NS = 16
NW = NC * NS
EPW = E // NW
B = 80
CH = EPW // B
RPT = NP // NS

def _sc_body(with_deg, x_hbm, src_hbm, dst_hbm, z2_hbm, z1_hbm,
             out_hbm, deg_hbm, src_v, dst_v, rows_v, ones_v,
             acc_s, deg_s, sem):
    c = lax.axis_index("c")
    s = lax.axis_index("s")
    wid = s * NC + c
    row0 = s * RPT

    pltpu.sync_copy(z2_hbm, acc_s.at[pl.ds(row0, RPT)])
    if with_deg:
        pltpu.sync_copy(z1_hbm, deg_s.at[pl.ds(row0, RPT)])
        for i in range(B // 16):
            ones_v[pl.ds(i * 16, 16)] = jnp.ones((16,), jnp.float32)

    pltpu.sync_copy(src_hbm.at[wid], src_v)
    pltpu.sync_copy(dst_hbm.at[wid], dst_v)
    plsc.subcore_barrier()

    def chunk(i, carry):
        pltpu.async_copy(x_hbm.at[src_v.at[i]], rows_v, sem).wait()
        pltpu.sync_copy(rows_v, acc_s.at[dst_v.at[i]], add=True)
        if with_deg:
            pltpu.sync_copy(ones_v, deg_s.at[dst_v.at[i]], add=True)
        return carry

    lax.fori_loop(0, CH, chunk, 0)
    plsc.subcore_barrier()

    out0 = c * NP + row0
    pltpu.sync_copy(acc_s.at[pl.ds(row0, RPT)], out_hbm.at[pl.ds(out0, RPT)])
    if with_deg:
        pltpu.sync_copy(deg_s.at[pl.ds(row0, RPT)], deg_hbm.at[pl.ds(out0, RPT)])


@functools.lru_cache(maxsize=None)
def _make_sc_pass(with_deg):
    mesh = plsc.VectorSubcoreMesh(core_axis_name="c", subcore_axis_name="s")
    out_type = [jax.ShapeDtypeStruct((NC * NP, D), jnp.float32)]
    if with_deg:
        out_type.append(jax.ShapeDtypeStruct((NC * NP,), jnp.float32))
    kern = functools.partial(
        pl.kernel,
        mesh=mesh,
        out_type=out_type,
        scratch_types=[
            pltpu.VMEM((CH, B), jnp.int32),
            pltpu.VMEM((CH, B), jnp.int32),
            pltpu.VMEM((B, D), jnp.float32),
            pltpu.VMEM((B,), jnp.float32),
            pltpu.VMEM_SHARED((NP, D), jnp.float32),
            pltpu.VMEM_SHARED((NP,), jnp.float32),
            pltpu.SemaphoreType.DMA,
        ],
    )

    if with_deg:
        @kern
        def sc_pass(x_hbm, src_hbm, dst_hbm, z2_hbm, z1_hbm, out_hbm, deg_hbm,
                    *scratch):
            _sc_body(True, x_hbm, src_hbm, dst_hbm, z2_hbm, z1_hbm,
                     out_hbm, deg_hbm, *scratch)
    else:
        @kern
        def sc_pass(x_hbm, src_hbm, dst_hbm, z2_hbm, out_hbm, *scratch):
            _sc_body(False, x_hbm, src_hbm, dst_hbm, z2_hbm, None,
                     out_hbm, None, *scratch)

    return sc_pass

BR = 1024
NB = NP // BR


def _dense_body(sa, sb, da, db, x, wl, wr, b, o, *, relu):
    deg = jnp.maximum(da[...] + db[...], 1.0)
    agg = (sa[...] + sb[...]) * (1.0 / deg)[:, None]
    y = jnp.dot(agg, wl[...], preferred_element_type=jnp.float32)
    y = y + jnp.dot(x[...], wr[...], preferred_element_type=jnp.float32)
    y = y + b[...]
    o[...] = jnp.maximum(y, 0.0) if relu else y


def _dense(summed, deg, xin, WlT, WrT, b, relu):
    return pl.pallas_call(
        functools.partial(_dense_body, relu=relu),
        grid=(NB,),
        in_specs=[
            pl.BlockSpec((BR, D), lambda i: (i, 0)),
            pl.BlockSpec((BR, D), lambda i: (i + NB, 0)),
            pl.BlockSpec((BR,), lambda i: (i,)),
            pl.BlockSpec((BR,), lambda i: (i + NB,)),
            pl.BlockSpec((BR, D), lambda i: (i, 0)),
            pl.BlockSpec((D, D), lambda i: (0, 0)),
            pl.BlockSpec((D, D), lambda i: (0, 0)),
            pl.BlockSpec((1, D), lambda i: (0, 0)),
        ],
        out_specs=pl.BlockSpec((BR, D), lambda i: (i, 0)),
        out_shape=jax.ShapeDtypeStruct((NP, D), jnp.float32),
    )(summed, summed, deg, deg, xin, WlT, WrT, b)


def kernel(x, edge_index, W1_l, b1, W1_r, W2_l, b2, W2_r):
    src = edge_index[0].astype(jnp.int32).reshape(NW, CH, B)
    dst = edge_index[1].astype(jnp.int32).reshape(NW, CH, B)
    z2 = jnp.zeros((RPT, D), jnp.float32)
    z1 = jnp.zeros((RPT,), jnp.float32)
    x_pad = jnp.pad(x, ((0, NP - N), (0, 0)))

    summed1, deg = _make_sc_pass(True)(x, src, dst, z2, z1)
    h = _dense(summed1, deg, x_pad, W1_l.T, W1_r.T, b1.reshape(1, D),
               relu=True)
    (summed2,) = _make_sc_pass(False)(h, src, dst, z2)
    out = _dense(summed2, deg, h, W2_l.T, W2_r.T, b2.reshape(1, D),
                 relu=False)
    return out[:N]
